# baseline (speedup 1.0000x reference)
import functools

import jax
import jax.numpy as jnp
from jax import lax
from jax.experimental import pallas as pl
from jax.experimental.pallas import tpu as pltpu
from jax.experimental.pallas import tpu_sc as plsc

_INFO = plsc.get_sparse_core_info()
_NC, _NS = _INFO.num_cores, _INFO.num_subcores

_L = 48
_P = 50
_D = 1024
_ROWS_PER_W = 8
_ACTIVE_W = _L // _ROWS_PER_W

_B = 1024
_BLK = 64


@functools.partial(
    pl.kernel,
    out_type=jax.ShapeDtypeStruct((_L, _D), jnp.float32),
    mesh=plsc.VectorSubcoreMesh(core_axis_name="c", subcore_axis_name="s"),
    scratch_types=[
        pltpu.VMEM((_ROWS_PER_W,), jnp.int32),
        pltpu.VMEM((_ROWS_PER_W, _D), jnp.float32),
        pltpu.SemaphoreType.DMA,
    ],
    compiler_params=pltpu.CompilerParams(use_tc_tiling_on_sc=True),
)
def _sc_gather(pe_hbm, idx_hbm, tok_hbm, sel_hbm, idx_v, rows_v, sem):
    del tok_hbm
    wid = lax.axis_index("s") * _NC + lax.axis_index("c")

    @pl.when(wid < _ACTIVE_W)
    def _():
        base = wid * _ROWS_PER_W
        pltpu.sync_copy(idx_hbm.at[pl.ds(base, _ROWS_PER_W)], idx_v)
        pltpu.async_copy(pe_hbm.at[idx_v], rows_v, sem).wait()
        pltpu.sync_copy(rows_v, sel_hbm.at[pl.ds(base, _ROWS_PER_W), :])


def _gather_sel_from_vmem(idx_ref, pe_ref, sel_ref):
    def body(l, carry):
        sel_ref[pl.ds(l, 1), :] = pe_ref[pl.ds(idx_ref[l], 1), :]
        return carry

    lax.fori_loop(0, _L, body, 0)


def _head1_body(idx_ref, pe_ref, x_ref, o_ref, tok_ref, sel_ref):
    _gather_sel_from_vmem(idx_ref, pe_ref, sel_ref)
    tok_ref[...] = jnp.zeros_like(tok_ref)
    o_ref[...] = x_ref[...] + sel_ref[...][None]


def _head2_body(idx_ref, pe_ref, prev_ref, x_ref, o_ref, sel_ref):
    _gather_sel_from_vmem(idx_ref, pe_ref, sel_ref)
    o_ref[...] = x_ref[...] + sel_ref[...][None]


def _tc_add_head1(x, pe, layer_indices):
    return pl.pallas_call(
        _head1_body,
        grid=(1,),
        in_specs=[
            pl.BlockSpec(memory_space=pltpu.MemorySpace.SMEM),
            pl.BlockSpec((_P, _D), lambda i: (0, 0)),
            pl.BlockSpec((_BLK, _L, _D), lambda i: (i, 0, 0)),
        ],
        out_specs=[
            pl.BlockSpec((_BLK, _L, _D), lambda i: (i, 0, 0)),
            pl.BlockSpec((8, 128), lambda i: (0, 0)),
        ],
        out_shape=[
            jax.ShapeDtypeStruct((_B, _L, _D), jnp.float32),
            jax.ShapeDtypeStruct((8, 128), jnp.float32),
        ],
        scratch_shapes=[pltpu.VMEM((_L, _D), jnp.float32)],
        compiler_params=pltpu.CompilerParams(
            dimension_semantics=("arbitrary",),
        ),
    )(layer_indices, pe, x)


def _tc_add_head2(x, pe, layer_indices, out1):
    return pl.pallas_call(
        _head2_body,
        grid=(1,),
        in_specs=[
            pl.BlockSpec(memory_space=pltpu.MemorySpace.SMEM),
            pl.BlockSpec((_P, _D), lambda i: (0, 0)),
            pl.BlockSpec(memory_space=pl.ANY),
            pl.BlockSpec((_BLK, _L, _D), lambda i: (i + 1, 0, 0)),
        ],
        out_specs=pl.BlockSpec((_BLK, _L, _D), lambda i: (i + 1, 0, 0)),
        out_shape=jax.ShapeDtypeStruct((_B, _L, _D), jnp.float32),
        input_output_aliases={2: 0},
        scratch_shapes=[pltpu.VMEM((_L, _D), jnp.float32)],
        compiler_params=pltpu.CompilerParams(
            dimension_semantics=("arbitrary",),
        ),
    )(layer_indices, pe, out1, x)


def _tail_body(sel_ref, prev_ref, x_ref, o_ref):
    o_ref[...] = x_ref[...] + sel_ref[...][None]


def _tc_add_tail(sel, out2, x):
    off = 2
    return pl.pallas_call(
        _tail_body,
        grid=(_B // _BLK - off,),
        in_specs=[
            pl.BlockSpec((_L, _D), lambda i: (0, 0)),
            pl.BlockSpec(memory_space=pl.ANY),
            pl.BlockSpec((_BLK, _L, _D), lambda i: (i + off, 0, 0)),
        ],
        out_specs=pl.BlockSpec((_BLK, _L, _D), lambda i: (i + off, 0, 0)),
        out_shape=jax.ShapeDtypeStruct((_B, _L, _D), jnp.float32),
        input_output_aliases={1: 0},
        compiler_params=pltpu.CompilerParams(
            dimension_semantics=("arbitrary",),
        ),
    )(sel, out2, x)


def kernel(x, pe, layer_indices):
    idx = layer_indices.astype(jnp.int32)
    out1, tok = _tc_add_head1(x, pe, idx)
    sel = _sc_gather(pe, idx, tok)
    out2 = _tc_add_head2(x, pe, idx, out1)
    return _tc_add_tail(sel, out2, x)


# speedup vs baseline: 1.0331x; 1.0331x over previous
import functools

import jax
import jax.numpy as jnp
from jax import lax
from jax.experimental import pallas as pl
from jax.experimental.pallas import tpu as pltpu
from jax.experimental.pallas import tpu_sc as plsc

_INFO = plsc.get_sparse_core_info()
_NC, _NS = _INFO.num_cores, _INFO.num_subcores

_L = 48
_P = 50
_D = 1024

_B = 1024
_B_HEAD = 128
_BLK = 64


@functools.partial(
    pl.kernel,
    out_type=jax.ShapeDtypeStruct((_L, _D), jnp.float32),
    mesh=plsc.VectorSubcoreMesh(core_axis_name="c", subcore_axis_name="s"),
    scratch_types=[
        pltpu.VMEM((_L,), jnp.int32),
        pltpu.VMEM((_L, _D), jnp.float32),
        pltpu.SemaphoreType.DMA,
    ],
    compiler_params=pltpu.CompilerParams(use_tc_tiling_on_sc=True),
)
def _sc_gather(pe_hbm, idx_hbm, sel_hbm, idx_v, rows_v, sem):
    wid = lax.axis_index("s") * _NC + lax.axis_index("c")

    @pl.when(wid == 0)
    def _():
        pltpu.sync_copy(idx_hbm, idx_v)
        pltpu.async_copy(pe_hbm.at[idx_v], rows_v, sem).wait()
        pltpu.sync_copy(rows_v, sel_hbm)


def _head_body(idx_ref, pe_ref, x_ref, o_ref, sel_ref):
    @pl.when(pl.program_id(0) == 0)
    def _():
        def body(l, carry):
            sel_ref[pl.ds(l, 1), :] = pe_ref[pl.ds(idx_ref[l], 1), :]
            return carry

        lax.fori_loop(0, _L, body, 0)

    o_ref[...] = x_ref[...] + sel_ref[...][None]


def _tc_add_head(x, pe, layer_indices):
    return pl.pallas_call(
        _head_body,
        grid=(_B_HEAD // _BLK,),
        in_specs=[
            pl.BlockSpec(memory_space=pltpu.MemorySpace.SMEM),
            pl.BlockSpec((_P, _D), lambda i: (0, 0)),
            pl.BlockSpec((_BLK, _L, _D), lambda i: (i, 0, 0)),
        ],
        out_specs=pl.BlockSpec((_BLK, _L, _D), lambda i: (i, 0, 0)),
        out_shape=jax.ShapeDtypeStruct((_B, _L, _D), jnp.float32),
        scratch_shapes=[pltpu.VMEM((_L, _D), jnp.float32)],
        compiler_params=pltpu.CompilerParams(
            dimension_semantics=("arbitrary",),
        ),
    )(layer_indices, pe, x)


def _tail_body(sel_ref, prev_ref, x_ref, o_ref):
    o_ref[...] = x_ref[...] + sel_ref[...][None]


def _tc_add_tail(sel, out1, x):
    off = _B_HEAD // _BLK
    return pl.pallas_call(
        _tail_body,
        grid=((_B - _B_HEAD) // _BLK,),
        in_specs=[
            pl.BlockSpec((_L, _D), lambda i: (0, 0)),
            pl.BlockSpec(memory_space=pl.ANY),
            pl.BlockSpec((_BLK, _L, _D), lambda i: (i + off, 0, 0)),
        ],
        out_specs=pl.BlockSpec((_BLK, _L, _D), lambda i: (i + off, 0, 0)),
        out_shape=jax.ShapeDtypeStruct((_B, _L, _D), jnp.float32),
        input_output_aliases={1: 0},
        compiler_params=pltpu.CompilerParams(
            dimension_semantics=("arbitrary",),
        ),
    )(sel, out1, x)


def kernel(x, pe, layer_indices):
    idx = layer_indices.astype(jnp.int32)
    sel = _sc_gather(pe, idx)
    out1 = _tc_add_head(x, pe, idx)
    return _tc_add_tail(sel, out1, x)
